# baseline (speedup 1.0000x reference)
import jax
import jax.numpy as jnp
from jax.experimental import pallas as pl
from jax.experimental.pallas import tpu as pltpu
from jax.experimental.pallas import tpu_sc as plsc

_MASK_TOKEN_ID = 8191
_P = 1024
_K = 16
_L = 16
_R = 256
_CHUNK = 4


def _stats_kernel(x_ref, pmax_ref, tok_ref):
    xb = x_ref[0]
    m = jnp.max(xb, axis=1, keepdims=True)
    e = jnp.exp(xb - m)
    s = jnp.sum(e, axis=1, keepdims=True)
    iota = jax.lax.broadcasted_iota(jnp.int32, xb.shape, 1)
    a = jnp.min(jnp.where(xb == m, iota, xb.shape[1]), axis=1, keepdims=True)
    pmax_ref[0] = 1.0 / s
    tok_ref[0] = a


def _sc_stats_kernel(logits_hbm, pmax_hbm, tok_hbm, rows_v, pstage_v, tstage_v,
                     red_f, red_i):
    _, s, v = logits_hbm.shape
    nvec = v // _L
    rows_per_worker = _R // 4
    wid = jax.lax.axis_index("s") * 2 + jax.lax.axis_index("c")
    b = wid // 4
    p0 = (wid & 3) * rows_per_worker
    iota = jax.lax.iota(jnp.int32, _L)

    def _allred(vv, scratch, op):
        for d in (8, 4, 2, 1):
            scratch[...] = vv
            vv = op(vv, plsc.load_gather(scratch, [iota ^ d]))
        return vv

    def chunk_body(cidx, _):
        seq0 = (s - _R) + p0 + cidx * _CHUNK
        pltpu.sync_copy(logits_hbm.at[b, pl.ds(seq0, _CHUNK)], rows_v)

        def row_body(r, _):
            def scan_max(j, carry):
                m, bj = carry
                xv = rows_v[r, pl.ds(j * _L, _L)]
                return jnp.maximum(m, xv), jnp.where(xv > m, j, bj)

            m, bj = jax.lax.fori_loop(
                0, nvec, scan_max,
                (jnp.full((_L,), -jnp.inf, jnp.float32),
                 jnp.zeros((_L,), jnp.int32)))
            gmax = _allred(m, red_f, jnp.maximum)
            cand = jnp.where(m == gmax, bj * _L + iota, v)
            bidx = _allred(cand, red_i, jnp.minimum)

            def scan_sum(j, acc):
                xv = rows_v[r, pl.ds(j * _L, _L)]
                return acc + jnp.exp(xv - gmax)

            acc = jax.lax.fori_loop(
                0, nvec, scan_sum, jnp.zeros((_L,), jnp.float32))
            ssum = _allred(acc, red_f, jnp.add)
            lane0 = iota == 0
            i_spl = jnp.zeros((_L,), jnp.int32) + (cidx * _CHUNK + r)
            plsc.store_scatter(pstage_v, [i_spl], 1.0 / ssum, mask=lane0)
            plsc.store_scatter(tstage_v, [i_spl], bidx, mask=lane0)
            return 0

        jax.lax.fori_loop(0, _CHUNK, row_body, 0)
        return 0

    jax.lax.fori_loop(0, rows_per_worker // _CHUNK, chunk_body, 0)
    pltpu.sync_copy(pstage_v, pmax_hbm.at[b, pl.ds(p0, rows_per_worker)])
    pltpu.sync_copy(tstage_v, tok_hbm.at[b, pl.ds(p0, rows_per_worker)])


def _topk_kernel(pmax_ref, tok_ref, shift_ref, probs_ref, suf_ref):
    p = pmax_ref[...]
    tok = tok_ref[...]
    shift = shift_ref[0, 0]
    b, ssuf = p.shape
    iota = jax.lax.broadcasted_iota(jnp.int32, p.shape, 1)
    colk = jax.lax.broadcasted_iota(jnp.int32, (b, _K), 1)
    sel = jnp.zeros((b, _K), jnp.int32)
    vals = jnp.zeros((b, _K), jnp.float32)
    for i in range(_K):
        m = jnp.max(p, axis=1, keepdims=True)
        cand = jnp.where(p == m, iota, ssuf)
        idx = jnp.min(cand, axis=1, keepdims=True)
        sel = jnp.where(colk == i, idx, sel)
        vals = jnp.where(colk == i, m, vals)
        p = jnp.where(iota == idx, -jnp.inf, p)
    probs_ref[...] = vals
    q = sel + shift
    rank = jnp.zeros_like(q)
    for j in range(_K):
        rank = rank + (q[:, j : j + 1] < q).astype(jnp.int32)
    out = jnp.full(p.shape, _MASK_TOKEN_ID, jnp.int32)
    for i in range(_K):
        pos = q[:, i : i + 1]
        t = jnp.sum(jnp.where(iota == pos, tok, 0), axis=1, keepdims=True)
        out = jnp.where(iota == rank[:, i : i + 1], t, out)
    suf_ref[...] = out


def kernel(logits, x, output_start_idx, k):
    b, s, v = logits.shape
    ssuf = s - _P
    ntc = ssuf - _R
    cs = 256
    pmax_tc, tok_tc = pl.pallas_call(
        _stats_kernel,
        grid=(b, ntc // cs),
        in_specs=[pl.BlockSpec((1, cs, v), lambda i, c: (i, c + _P // cs, 0))],
        out_specs=[
            pl.BlockSpec((1, cs, 1), lambda i, c: (i, c, 0)),
            pl.BlockSpec((1, cs, 1), lambda i, c: (i, c, 0)),
        ],
        out_shape=[
            jax.ShapeDtypeStruct((b, ntc, 1), jnp.float32),
            jax.ShapeDtypeStruct((b, ntc, 1), jnp.int32),
        ],
    )(logits)
    rows_per_worker = _R // 4
    sc_stats = pl.kernel(
        _sc_stats_kernel,
        out_type=[
            jax.ShapeDtypeStruct((b, _R), jnp.float32),
            jax.ShapeDtypeStruct((b, _R), jnp.int32),
        ],
        scratch_types=[
            pltpu.VMEM((_CHUNK, v), jnp.float32),
            pltpu.VMEM((rows_per_worker,), jnp.float32),
            pltpu.VMEM((rows_per_worker,), jnp.int32),
            pltpu.VMEM((_L,), jnp.float32),
            pltpu.VMEM((_L,), jnp.int32),
        ],
        mesh=plsc.VectorSubcoreMesh(core_axis_name="c", subcore_axis_name="s"),
        compiler_params=pltpu.CompilerParams(needs_layout_passes=False),
    )
    pmax_sc, tok_sc = sc_stats(logits)
    pmax2 = jnp.concatenate([pmax_tc.reshape(b, ntc), pmax_sc], axis=1)
    tok2 = jnp.concatenate([tok_tc.reshape(b, ntc), tok_sc], axis=1)
    shift = (jnp.asarray(output_start_idx, jnp.int32) - _P
             + jnp.asarray(k, jnp.int32) - _K).reshape(1, 1)
    probs, out_suf = pl.pallas_call(
        _topk_kernel,
        out_shape=[
            jax.ShapeDtypeStruct((b, _K), jnp.float32),
            jax.ShapeDtypeStruct((b, ssuf), jnp.int32),
        ],
    )(pmax2, tok2, shift)
    out = jnp.concatenate([x[:, :_P], out_suf], axis=1)
    return out, probs


# speedup vs baseline: 2.4162x; 2.4162x over previous
import jax
import jax.numpy as jnp
from jax.experimental import pallas as pl
from jax.experimental.pallas import tpu as pltpu
from jax.experimental.pallas import tpu_sc as plsc

_MASK_TOKEN_ID = 8191
_P = 1024
_K = 16
_L = 16
_R = 256
_CHUNK = 4


def _stats_kernel(x_ref, pmax_ref, tok_ref):
    xb = x_ref[0]
    m = jnp.max(xb, axis=1, keepdims=True)
    e = jnp.exp(xb - m)
    s = jnp.sum(e, axis=1, keepdims=True)
    iota = jax.lax.broadcasted_iota(jnp.int32, xb.shape, 1)
    a = jnp.min(jnp.where(xb == m, iota, xb.shape[1]), axis=1, keepdims=True)
    pmax_ref[0] = 1.0 / s
    tok_ref[0] = a


def _sc_stats_kernel(logits_hbm, pmax_hbm, tok_hbm, rows_v, pstage_v, tstage_v,
                     red_f, red_i):
    _, s, v = logits_hbm.shape
    nvec = v // _L
    rows_per_worker = _R // 4
    wid = jax.lax.axis_index("s") * 2 + jax.lax.axis_index("c")
    b = wid // 4
    p0 = (wid & 3) * rows_per_worker
    iota = jax.lax.iota(jnp.int32, _L)

    def _allred(vv, scratch, op):
        for d in (8, 4, 2, 1):
            scratch[...] = vv
            vv = op(vv, plsc.load_gather(scratch, [iota ^ d]))
        return vv

    def chunk_body(cidx, _):
        seq0 = (s - _R) + p0 + cidx * _CHUNK
        pltpu.sync_copy(logits_hbm.at[b, pl.ds(seq0, _CHUNK)], rows_v)

        def row_body(r, _):
            def scan_max(j, carry):
                m, bj = carry
                xv = rows_v[r, pl.ds(j * _L, _L)]
                return jnp.maximum(m, xv), jnp.where(xv > m, j, bj)

            m, bj = jax.lax.fori_loop(
                0, nvec, scan_max,
                (jnp.full((_L,), -jnp.inf, jnp.float32),
                 jnp.zeros((_L,), jnp.int32)), unroll=8)
            gmax = _allred(m, red_f, jnp.maximum)
            cand = jnp.where(m == gmax, bj * _L + iota, v)
            bidx = _allred(cand, red_i, jnp.minimum)

            def scan_sum(j, acc):
                xv = rows_v[r, pl.ds(j * _L, _L)]
                return acc + jnp.exp(xv - gmax)

            acc = jax.lax.fori_loop(
                0, nvec, scan_sum, jnp.zeros((_L,), jnp.float32), unroll=8)
            ssum = _allred(acc, red_f, jnp.add)
            lane0 = iota == 0
            i_spl = jnp.zeros((_L,), jnp.int32) + (cidx * _CHUNK + r)
            plsc.store_scatter(pstage_v, [i_spl], 1.0 / ssum, mask=lane0)
            plsc.store_scatter(tstage_v, [i_spl], bidx, mask=lane0)
            return 0

        jax.lax.fori_loop(0, _CHUNK, row_body, 0)
        return 0

    jax.lax.fori_loop(0, rows_per_worker // _CHUNK, chunk_body, 0)
    pltpu.sync_copy(pstage_v, pmax_hbm.at[b, pl.ds(p0, rows_per_worker)])
    pltpu.sync_copy(tstage_v, tok_hbm.at[b, pl.ds(p0, rows_per_worker)])


def _topk_kernel(pmax_ref, tok_ref, shift_ref, probs_ref, suf_ref):
    p = pmax_ref[...]
    tok = tok_ref[...]
    shift = shift_ref[0, 0]
    b, ssuf = p.shape
    iota = jax.lax.broadcasted_iota(jnp.int32, p.shape, 1)
    colk = jax.lax.broadcasted_iota(jnp.int32, (b, _K), 1)
    sel = jnp.zeros((b, _K), jnp.int32)
    vals = jnp.zeros((b, _K), jnp.float32)
    for i in range(_K):
        m = jnp.max(p, axis=1, keepdims=True)
        cand = jnp.where(p == m, iota, ssuf)
        idx = jnp.min(cand, axis=1, keepdims=True)
        sel = jnp.where(colk == i, idx, sel)
        vals = jnp.where(colk == i, m, vals)
        p = jnp.where(iota == idx, -jnp.inf, p)
    probs_ref[...] = vals
    q = sel + shift
    rank = jnp.zeros_like(q)
    for j in range(_K):
        rank = rank + (q[:, j : j + 1] < q).astype(jnp.int32)
    out = jnp.full(p.shape, _MASK_TOKEN_ID, jnp.int32)
    for i in range(_K):
        pos = q[:, i : i + 1]
        t = jnp.sum(jnp.where(iota == pos, tok, 0), axis=1, keepdims=True)
        out = jnp.where(iota == rank[:, i : i + 1], t, out)
    suf_ref[...] = out


def kernel(logits, x, output_start_idx, k):
    b, s, v = logits.shape
    ssuf = s - _P
    ntc = ssuf - _R
    cs = 256
    pmax_tc, tok_tc = pl.pallas_call(
        _stats_kernel,
        grid=(b, ntc // cs),
        in_specs=[pl.BlockSpec((1, cs, v), lambda i, c: (i, c + _P // cs, 0))],
        out_specs=[
            pl.BlockSpec((1, cs, 1), lambda i, c: (i, c, 0)),
            pl.BlockSpec((1, cs, 1), lambda i, c: (i, c, 0)),
        ],
        out_shape=[
            jax.ShapeDtypeStruct((b, ntc, 1), jnp.float32),
            jax.ShapeDtypeStruct((b, ntc, 1), jnp.int32),
        ],
    )(logits)
    rows_per_worker = _R // 4
    sc_stats = pl.kernel(
        _sc_stats_kernel,
        out_type=[
            jax.ShapeDtypeStruct((b, _R), jnp.float32),
            jax.ShapeDtypeStruct((b, _R), jnp.int32),
        ],
        scratch_types=[
            pltpu.VMEM((_CHUNK, v), jnp.float32),
            pltpu.VMEM((rows_per_worker,), jnp.float32),
            pltpu.VMEM((rows_per_worker,), jnp.int32),
            pltpu.VMEM((_L,), jnp.float32),
            pltpu.VMEM((_L,), jnp.int32),
        ],
        mesh=plsc.VectorSubcoreMesh(core_axis_name="c", subcore_axis_name="s"),
        compiler_params=pltpu.CompilerParams(needs_layout_passes=False),
    )
    pmax_sc, tok_sc = sc_stats(logits)
    pmax2 = jnp.concatenate([pmax_tc.reshape(b, ntc), pmax_sc], axis=1)
    tok2 = jnp.concatenate([tok_tc.reshape(b, ntc), tok_sc], axis=1)
    shift = (jnp.asarray(output_start_idx, jnp.int32) - _P
             + jnp.asarray(k, jnp.int32) - _K).reshape(1, 1)
    probs, out_suf = pl.pallas_call(
        _topk_kernel,
        out_shape=[
            jax.ShapeDtypeStruct((b, _K), jnp.float32),
            jax.ShapeDtypeStruct((b, ssuf), jnp.int32),
        ],
    )(pmax2, tok2, shift)
    out = jnp.concatenate([x[:, :_P], out_suf], axis=1)
    return out, probs


# speedup vs baseline: 2.4660x; 1.0206x over previous
import jax
import jax.numpy as jnp
from jax.experimental import pallas as pl
from jax.experimental.pallas import tpu as pltpu
from jax.experimental.pallas import tpu_sc as plsc

_MASK_TOKEN_ID = 8191
_P = 1024
_K = 16
_L = 16
_R = 256
_CHUNK = 8


def _stats_kernel(x_ref, pmax_ref, tok_ref):
    xb = x_ref[0]
    m = jnp.max(xb, axis=1, keepdims=True)
    e = jnp.exp(xb - m)
    s = jnp.sum(e, axis=1, keepdims=True)
    iota = jax.lax.broadcasted_iota(jnp.int32, xb.shape, 1)
    a = jnp.min(jnp.where(xb == m, iota, xb.shape[1]), axis=1, keepdims=True)
    pmax_ref[0] = 1.0 / s
    tok_ref[0] = a


def _sc_stats_kernel(logits_hbm, pmax_hbm, tok_hbm, rows_v, pstage_v, tstage_v,
                     red_f, red_i):
    _, s, v = logits_hbm.shape
    nvec = v // _L
    rows_per_worker = _R // 4
    wid = jax.lax.axis_index("s") * 2 + jax.lax.axis_index("c")
    b = wid // 4
    p0 = (wid & 3) * rows_per_worker
    iota = jax.lax.iota(jnp.int32, _L)

    def _allred(vv, scratch, op):
        for d in (8, 4, 2, 1):
            scratch[...] = vv
            vv = op(vv, plsc.load_gather(scratch, [iota ^ d]))
        return vv

    def chunk_body(cidx, _):
        seq0 = (s - _R) + p0 + cidx * _CHUNK
        pltpu.sync_copy(logits_hbm.at[b, pl.ds(seq0, _CHUNK)], rows_v)

        def row_body(r, _):
            def scan_max(j, carry):
                m, bj = carry
                xv = rows_v[r, pl.ds(j * _L, _L)]
                return jnp.maximum(m, xv), jnp.where(xv > m, j, bj)

            m, bj = jax.lax.fori_loop(
                0, nvec, scan_max,
                (jnp.full((_L,), -jnp.inf, jnp.float32),
                 jnp.zeros((_L,), jnp.int32)), unroll=16)
            gmax = _allred(m, red_f, jnp.maximum)
            cand = jnp.where(m == gmax, bj * _L + iota, v)
            bidx = _allred(cand, red_i, jnp.minimum)

            def scan_sum(j, acc):
                xv = rows_v[r, pl.ds(j * _L, _L)]
                return acc + jnp.exp(xv - gmax)

            acc = jax.lax.fori_loop(
                0, nvec, scan_sum, jnp.zeros((_L,), jnp.float32), unroll=16)
            ssum = _allred(acc, red_f, jnp.add)
            lane0 = iota == 0
            i_spl = jnp.zeros((_L,), jnp.int32) + (cidx * _CHUNK + r)
            plsc.store_scatter(pstage_v, [i_spl], 1.0 / ssum, mask=lane0)
            plsc.store_scatter(tstage_v, [i_spl], bidx, mask=lane0)
            return 0

        jax.lax.fori_loop(0, _CHUNK, row_body, 0)
        return 0

    jax.lax.fori_loop(0, rows_per_worker // _CHUNK, chunk_body, 0)
    pltpu.sync_copy(pstage_v, pmax_hbm.at[b, pl.ds(p0, rows_per_worker)])
    pltpu.sync_copy(tstage_v, tok_hbm.at[b, pl.ds(p0, rows_per_worker)])


def _topk_kernel(pmax_ref, tok_ref, shift_ref, probs_ref, suf_ref):
    p = pmax_ref[...]
    tok = tok_ref[...]
    shift = shift_ref[0, 0]
    b, ssuf = p.shape
    iota = jax.lax.broadcasted_iota(jnp.int32, p.shape, 1)
    colk = jax.lax.broadcasted_iota(jnp.int32, (b, _K), 1)
    sel = jnp.zeros((b, _K), jnp.int32)
    vals = jnp.zeros((b, _K), jnp.float32)
    for i in range(_K):
        m = jnp.max(p, axis=1, keepdims=True)
        cand = jnp.where(p == m, iota, ssuf)
        idx = jnp.min(cand, axis=1, keepdims=True)
        sel = jnp.where(colk == i, idx, sel)
        vals = jnp.where(colk == i, m, vals)
        p = jnp.where(iota == idx, -jnp.inf, p)
    probs_ref[...] = vals
    q = sel + shift
    rank = jnp.zeros_like(q)
    for j in range(_K):
        rank = rank + (q[:, j : j + 1] < q).astype(jnp.int32)
    out = jnp.full(p.shape, _MASK_TOKEN_ID, jnp.int32)
    for i in range(_K):
        pos = q[:, i : i + 1]
        t = jnp.sum(jnp.where(iota == pos, tok, 0), axis=1, keepdims=True)
        out = jnp.where(iota == rank[:, i : i + 1], t, out)
    suf_ref[...] = out


def kernel(logits, x, output_start_idx, k):
    b, s, v = logits.shape
    ssuf = s - _P
    ntc = ssuf - _R
    cs = 256
    pmax_tc, tok_tc = pl.pallas_call(
        _stats_kernel,
        grid=(b, ntc // cs),
        in_specs=[pl.BlockSpec((1, cs, v), lambda i, c: (i, c + _P // cs, 0))],
        out_specs=[
            pl.BlockSpec((1, cs, 1), lambda i, c: (i, c, 0)),
            pl.BlockSpec((1, cs, 1), lambda i, c: (i, c, 0)),
        ],
        out_shape=[
            jax.ShapeDtypeStruct((b, ntc, 1), jnp.float32),
            jax.ShapeDtypeStruct((b, ntc, 1), jnp.int32),
        ],
    )(logits)
    rows_per_worker = _R // 4
    sc_stats = pl.kernel(
        _sc_stats_kernel,
        out_type=[
            jax.ShapeDtypeStruct((b, _R), jnp.float32),
            jax.ShapeDtypeStruct((b, _R), jnp.int32),
        ],
        scratch_types=[
            pltpu.VMEM((_CHUNK, v), jnp.float32),
            pltpu.VMEM((rows_per_worker,), jnp.float32),
            pltpu.VMEM((rows_per_worker,), jnp.int32),
            pltpu.VMEM((_L,), jnp.float32),
            pltpu.VMEM((_L,), jnp.int32),
        ],
        mesh=plsc.VectorSubcoreMesh(core_axis_name="c", subcore_axis_name="s"),
        compiler_params=pltpu.CompilerParams(needs_layout_passes=False),
    )
    pmax_sc, tok_sc = sc_stats(logits)
    pmax2 = jnp.concatenate([pmax_tc.reshape(b, ntc), pmax_sc], axis=1)
    tok2 = jnp.concatenate([tok_tc.reshape(b, ntc), tok_sc], axis=1)
    shift = (jnp.asarray(output_start_idx, jnp.int32) - _P
             + jnp.asarray(k, jnp.int32) - _K).reshape(1, 1)
    probs, out_suf = pl.pallas_call(
        _topk_kernel,
        out_shape=[
            jax.ShapeDtypeStruct((b, _K), jnp.float32),
            jax.ShapeDtypeStruct((b, ssuf), jnp.int32),
        ],
    )(pmax2, tok2, shift)
    out = jnp.concatenate([x[:, :_P], out_suf], axis=1)
    return out, probs


# speedup vs baseline: 2.6191x; 1.0621x over previous
import jax
import jax.numpy as jnp
from jax.experimental import pallas as pl
from jax.experimental.pallas import tpu as pltpu
from jax.experimental.pallas import tpu_sc as plsc

_MASK_TOKEN_ID = 8191
_P = 1024
_K = 16
_L = 16
_R = 256
_CHUNK = 8


def _stats_kernel(x_ref, pmax_ref, tok_ref):
    xb = x_ref[0]
    m = jnp.max(xb, axis=1, keepdims=True)
    e = jnp.exp(xb - m)
    s = jnp.sum(e, axis=1, keepdims=True)
    iota = jax.lax.broadcasted_iota(jnp.int32, xb.shape, 1)
    a = jnp.min(jnp.where(xb == m, iota, xb.shape[1]), axis=1, keepdims=True)
    pmax_ref[0] = 1.0 / s
    tok_ref[0] = a


def _sc_stats_kernel(logits_hbm, pmax_hbm, tok_hbm, rows_v, pstage_v, tstage_v,
                     red_f, red_i):
    _, s, v = logits_hbm.shape
    nvec = v // _L
    rows_per_worker = _R // 4
    wid = jax.lax.axis_index("s") * 2 + jax.lax.axis_index("c")
    b = wid // 4
    p0 = (wid & 3) * rows_per_worker
    iota = jax.lax.iota(jnp.int32, _L)

    def _allred(vv, scratch, op):
        for d in (8, 4, 2, 1):
            scratch[...] = vv
            vv = op(vv, plsc.load_gather(scratch, [iota ^ d]))
        return vv

    def chunk_body(cidx, _):
        seq0 = (s - _R) + p0 + cidx * _CHUNK
        pltpu.sync_copy(logits_hbm.at[b, pl.ds(seq0, _CHUNK)], rows_v)

        def row_body(r, _):
            def scan_max(j, carry):
                ms, bjs = carry
                new_ms, new_bjs = [], []
                for q in range(4):
                    xv = rows_v[r, pl.ds((j * 4 + q) * _L, _L)]
                    new_ms.append(jnp.maximum(ms[q], xv))
                    new_bjs.append(jnp.where(xv > ms[q], j, bjs[q]))
                return tuple(new_ms), tuple(new_bjs)

            ms, bjs = jax.lax.fori_loop(
                0, nvec // 4, scan_max,
                (tuple(jnp.full((_L,), -jnp.inf, jnp.float32) for _ in range(4)),
                 tuple(jnp.zeros((_L,), jnp.int32) for _ in range(4))),
                unroll=4)
            m01 = jnp.maximum(ms[0], ms[1])
            m23 = jnp.maximum(ms[2], ms[3])
            gmax = _allred(jnp.maximum(m01, m23), red_f, jnp.maximum)
            cand = jnp.full((_L,), v, jnp.int32)
            for q in range(4):
                idxq = (bjs[q] * 4 + q) * _L + iota
                cand = jnp.minimum(cand, jnp.where(ms[q] == gmax, idxq, v))
            bidx = _allred(cand, red_i, jnp.minimum)

            def scan_sum(j, accs):
                new = []
                for q in range(4):
                    xv = rows_v[r, pl.ds((j * 4 + q) * _L, _L)]
                    new.append(accs[q] + jnp.exp(xv - gmax))
                return tuple(new)

            accs = jax.lax.fori_loop(
                0, nvec // 4, scan_sum,
                tuple(jnp.zeros((_L,), jnp.float32) for _ in range(4)),
                unroll=4)
            ssum = _allred((accs[0] + accs[1]) + (accs[2] + accs[3]),
                           red_f, jnp.add)
            lane0 = iota == 0
            i_spl = jnp.zeros((_L,), jnp.int32) + (cidx * _CHUNK + r)
            plsc.store_scatter(pstage_v, [i_spl], 1.0 / ssum, mask=lane0)
            plsc.store_scatter(tstage_v, [i_spl], bidx, mask=lane0)
            return 0

        jax.lax.fori_loop(0, _CHUNK, row_body, 0)
        return 0

    jax.lax.fori_loop(0, rows_per_worker // _CHUNK, chunk_body, 0)
    pltpu.sync_copy(pstage_v, pmax_hbm.at[b, pl.ds(p0, rows_per_worker)])
    pltpu.sync_copy(tstage_v, tok_hbm.at[b, pl.ds(p0, rows_per_worker)])


def _topk_kernel(pmax_ref, tok_ref, shift_ref, probs_ref, suf_ref):
    p = pmax_ref[...]
    tok = tok_ref[...]
    shift = shift_ref[0, 0]
    b, ssuf = p.shape
    iota = jax.lax.broadcasted_iota(jnp.int32, p.shape, 1)
    colk = jax.lax.broadcasted_iota(jnp.int32, (b, _K), 1)
    sel = jnp.zeros((b, _K), jnp.int32)
    vals = jnp.zeros((b, _K), jnp.float32)
    for i in range(_K):
        m = jnp.max(p, axis=1, keepdims=True)
        cand = jnp.where(p == m, iota, ssuf)
        idx = jnp.min(cand, axis=1, keepdims=True)
        sel = jnp.where(colk == i, idx, sel)
        vals = jnp.where(colk == i, m, vals)
        p = jnp.where(iota == idx, -jnp.inf, p)
    probs_ref[...] = vals
    q = sel + shift
    rank = jnp.zeros_like(q)
    for j in range(_K):
        rank = rank + (q[:, j : j + 1] < q).astype(jnp.int32)
    out = jnp.full(p.shape, _MASK_TOKEN_ID, jnp.int32)
    for i in range(_K):
        pos = q[:, i : i + 1]
        t = jnp.sum(jnp.where(iota == pos, tok, 0), axis=1, keepdims=True)
        out = jnp.where(iota == rank[:, i : i + 1], t, out)
    suf_ref[...] = out


def kernel(logits, x, output_start_idx, k):
    b, s, v = logits.shape
    ssuf = s - _P
    ntc = ssuf - _R
    cs = 256
    pmax_tc, tok_tc = pl.pallas_call(
        _stats_kernel,
        grid=(b, ntc // cs),
        in_specs=[pl.BlockSpec((1, cs, v), lambda i, c: (i, c + _P // cs, 0))],
        out_specs=[
            pl.BlockSpec((1, cs, 1), lambda i, c: (i, c, 0)),
            pl.BlockSpec((1, cs, 1), lambda i, c: (i, c, 0)),
        ],
        out_shape=[
            jax.ShapeDtypeStruct((b, ntc, 1), jnp.float32),
            jax.ShapeDtypeStruct((b, ntc, 1), jnp.int32),
        ],
    )(logits)
    rows_per_worker = _R // 4
    sc_stats = pl.kernel(
        _sc_stats_kernel,
        out_type=[
            jax.ShapeDtypeStruct((b, _R), jnp.float32),
            jax.ShapeDtypeStruct((b, _R), jnp.int32),
        ],
        scratch_types=[
            pltpu.VMEM((_CHUNK, v), jnp.float32),
            pltpu.VMEM((rows_per_worker,), jnp.float32),
            pltpu.VMEM((rows_per_worker,), jnp.int32),
            pltpu.VMEM((_L,), jnp.float32),
            pltpu.VMEM((_L,), jnp.int32),
        ],
        mesh=plsc.VectorSubcoreMesh(core_axis_name="c", subcore_axis_name="s"),
        compiler_params=pltpu.CompilerParams(needs_layout_passes=False),
    )
    pmax_sc, tok_sc = sc_stats(logits)
    pmax2 = jnp.concatenate([pmax_tc.reshape(b, ntc), pmax_sc], axis=1)
    tok2 = jnp.concatenate([tok_tc.reshape(b, ntc), tok_sc], axis=1)
    shift = (jnp.asarray(output_start_idx, jnp.int32) - _P
             + jnp.asarray(k, jnp.int32) - _K).reshape(1, 1)
    probs, out_suf = pl.pallas_call(
        _topk_kernel,
        out_shape=[
            jax.ShapeDtypeStruct((b, _K), jnp.float32),
            jax.ShapeDtypeStruct((b, ssuf), jnp.int32),
        ],
    )(pmax2, tok2, shift)
    out = jnp.concatenate([x[:, :_P], out_suf], axis=1)
    return out, probs


# speedup vs baseline: 2.7180x; 1.0377x over previous
import jax
import jax.numpy as jnp
from jax.experimental import pallas as pl
from jax.experimental.pallas import tpu as pltpu
from jax.experimental.pallas import tpu_sc as plsc

_MASK_TOKEN_ID = 8191
_P = 1024
_K = 16
_L = 16
_R = 256
_CHUNK = 8


def _stats_kernel(x_ref, pmax_ref, tok_ref):
    xb = x_ref[0]
    m = jnp.max(xb, axis=1, keepdims=True)
    e = jnp.exp(xb - m)
    s = jnp.sum(e, axis=1, keepdims=True)
    iota = jax.lax.broadcasted_iota(jnp.int32, xb.shape, 1)
    a = jnp.min(jnp.where(xb == m, iota, xb.shape[1]), axis=1, keepdims=True)
    pmax_ref[0] = 1.0 / s
    tok_ref[0] = a


def _sc_stats_kernel(logits_hbm, pmax_hbm, tok_hbm, rows_v, pstage_v, tstage_v,
                     red_f, red_i):
    _, s, v = logits_hbm.shape
    nvec = v // _L
    rows_per_worker = _R // 4
    wid = jax.lax.axis_index("s") * 2 + jax.lax.axis_index("c")
    b = wid // 4
    p0 = (wid & 3) * rows_per_worker
    iota = jax.lax.iota(jnp.int32, _L)

    def _allred(vv, scratch, op):
        for d in (8, 4, 2, 1):
            scratch[...] = vv
            vv = op(vv, plsc.load_gather(scratch, [iota ^ d]))
        return vv

    def chunk_body(cidx, _):
        seq0 = (s - _R) + p0 + cidx * _CHUNK
        pltpu.sync_copy(logits_hbm.at[b, pl.ds(seq0, _CHUNK)], rows_v)

        def row_body(r, _):
            def scan_max(j, carry):
                ms, bjs = carry
                new_ms, new_bjs = [], []
                for q in range(4):
                    xv = rows_v[r, pl.ds((j * 4 + q) * _L, _L)]
                    new_ms.append(jnp.maximum(ms[q], xv))
                    new_bjs.append(jnp.where(xv > ms[q], j, bjs[q]))
                return tuple(new_ms), tuple(new_bjs)

            ms, bjs = jax.lax.fori_loop(
                0, nvec // 4, scan_max,
                (tuple(jnp.full((_L,), -jnp.inf, jnp.float32) for _ in range(4)),
                 tuple(jnp.zeros((_L,), jnp.int32) for _ in range(4))),
                unroll=4)
            m01 = jnp.maximum(ms[0], ms[1])
            m23 = jnp.maximum(ms[2], ms[3])
            gmax = _allred(jnp.maximum(m01, m23), red_f, jnp.maximum)
            cand = jnp.full((_L,), v, jnp.int32)
            for q in range(4):
                idxq = (bjs[q] * 4 + q) * _L + iota
                cand = jnp.minimum(cand, jnp.where(ms[q] == gmax, idxq, v))
            bidx = _allred(cand, red_i, jnp.minimum)

            def scan_sum(j, accs):
                new = []
                for q in range(4):
                    xv = rows_v[r, pl.ds((j * 4 + q) * _L, _L)]
                    new.append(accs[q] + jnp.exp(xv - gmax))
                return tuple(new)

            accs = jax.lax.fori_loop(
                0, nvec // 4, scan_sum,
                tuple(jnp.zeros((_L,), jnp.float32) for _ in range(4)),
                unroll=4)
            ssum = _allred((accs[0] + accs[1]) + (accs[2] + accs[3]),
                           red_f, jnp.add)
            lane0 = iota == 0
            i_spl = jnp.zeros((_L,), jnp.int32) + (cidx * _CHUNK + r)
            plsc.store_scatter(pstage_v, [i_spl], 1.0 / ssum, mask=lane0)
            plsc.store_scatter(tstage_v, [i_spl], bidx, mask=lane0)
            return 0

        jax.lax.fori_loop(0, _CHUNK, row_body, 0)
        return 0

    jax.lax.fori_loop(0, rows_per_worker // _CHUNK, chunk_body, 0)
    pltpu.sync_copy(pstage_v, pmax_hbm.at[b, pl.ds(p0, rows_per_worker)])
    pltpu.sync_copy(tstage_v, tok_hbm.at[b, pl.ds(p0, rows_per_worker)])


def _topk_kernel(ptc_ref, psc_ref, ttc_ref, tsc_ref, shift_ref, x_ref,
                 probs_ref, out_ref):
    p = jnp.concatenate([ptc_ref[...][:, :, 0], psc_ref[...]], axis=1)
    tok = jnp.concatenate([ttc_ref[...][:, :, 0], tsc_ref[...]], axis=1)
    shift = shift_ref[0, 0]
    b, ssuf = p.shape
    iota = jax.lax.broadcasted_iota(jnp.int32, p.shape, 1)
    colk = jax.lax.broadcasted_iota(jnp.int32, (b, _K), 1)
    sel = jnp.zeros((b, _K), jnp.int32)
    vals = jnp.zeros((b, _K), jnp.float32)
    for i in range(_K):
        m = jnp.max(p, axis=1, keepdims=True)
        cand = jnp.where(p == m, iota, ssuf)
        idx = jnp.min(cand, axis=1, keepdims=True)
        sel = jnp.where(colk == i, idx, sel)
        vals = jnp.where(colk == i, m, vals)
        p = jnp.where(iota == idx, -jnp.inf, p)
    probs_ref[...] = vals
    q = sel + shift
    rank = jnp.zeros_like(q)
    for j in range(_K):
        rank = rank + (q[:, j : j + 1] < q).astype(jnp.int32)
    out = jnp.full(p.shape, _MASK_TOKEN_ID, jnp.int32)
    for i in range(_K):
        pos = q[:, i : i + 1]
        t = jnp.sum(jnp.where(iota == pos, tok, 0), axis=1, keepdims=True)
        out = jnp.where(iota == rank[:, i : i + 1], t, out)
    out_ref[:, :_P] = x_ref[:, :_P]
    out_ref[:, _P:] = out


def kernel(logits, x, output_start_idx, k):
    b, s, v = logits.shape
    ssuf = s - _P
    ntc = ssuf - _R
    cs = 256
    pmax_tc, tok_tc = pl.pallas_call(
        _stats_kernel,
        grid=(b, ntc // cs),
        in_specs=[pl.BlockSpec((1, cs, v), lambda i, c: (i, c + _P // cs, 0))],
        out_specs=[
            pl.BlockSpec((1, cs, 1), lambda i, c: (i, c, 0)),
            pl.BlockSpec((1, cs, 1), lambda i, c: (i, c, 0)),
        ],
        out_shape=[
            jax.ShapeDtypeStruct((b, ntc, 1), jnp.float32),
            jax.ShapeDtypeStruct((b, ntc, 1), jnp.int32),
        ],
    )(logits)
    rows_per_worker = _R // 4
    sc_stats = pl.kernel(
        _sc_stats_kernel,
        out_type=[
            jax.ShapeDtypeStruct((b, _R), jnp.float32),
            jax.ShapeDtypeStruct((b, _R), jnp.int32),
        ],
        scratch_types=[
            pltpu.VMEM((_CHUNK, v), jnp.float32),
            pltpu.VMEM((rows_per_worker,), jnp.float32),
            pltpu.VMEM((rows_per_worker,), jnp.int32),
            pltpu.VMEM((_L,), jnp.float32),
            pltpu.VMEM((_L,), jnp.int32),
        ],
        mesh=plsc.VectorSubcoreMesh(core_axis_name="c", subcore_axis_name="s"),
        compiler_params=pltpu.CompilerParams(needs_layout_passes=False),
    )
    pmax_sc, tok_sc = sc_stats(logits)
    shift = (jnp.asarray(output_start_idx, jnp.int32) - _P
             + jnp.asarray(k, jnp.int32) - _K).reshape(1, 1)
    probs, out = pl.pallas_call(
        _topk_kernel,
        out_shape=[
            jax.ShapeDtypeStruct((b, _K), jnp.float32),
            jax.ShapeDtypeStruct((b, s), jnp.int32),
        ],
    )(pmax_tc, pmax_sc, tok_tc, tok_sc, shift, x)
    return out, probs


# speedup vs baseline: 2.7906x; 1.0267x over previous
import jax
import jax.numpy as jnp
from jax.experimental import pallas as pl
from jax.experimental.pallas import tpu as pltpu
from jax.experimental.pallas import tpu_sc as plsc

_MASK_TOKEN_ID = 8191
_P = 1024
_K = 16
_L = 16
_BSC = 2
_CHUNK = 8


def _stats_kernel(x_ref, pmax_ref, tok_ref):
    xb = x_ref[0]
    m = jnp.max(xb, axis=1, keepdims=True)
    e = jnp.exp(xb - m)
    s = jnp.sum(e, axis=1, keepdims=True)
    iota = jax.lax.broadcasted_iota(jnp.int32, xb.shape, 1)
    a = jnp.min(jnp.where(xb == m, iota, xb.shape[1]), axis=1, keepdims=True)
    pmax_ref[0] = 1.0 / s
    tok_ref[0] = a


def _sc_stats_kernel(logits_hbm, pmax_hbm, tok_hbm, rows_v, pstage_v, tstage_v,
                     red_f, red_i):
    _, s, v = logits_hbm.shape
    nvec = v // _L
    rows_per_worker = (_BSC * (s - _P)) // 32
    wid = jax.lax.axis_index("s") * 2 + jax.lax.axis_index("c")
    b = (8 - _BSC) + wid // (32 // _BSC)
    p0 = (wid % (32 // _BSC)) * rows_per_worker
    iota = jax.lax.iota(jnp.int32, _L)

    def _allred(vv, scratch, op):
        for d in (8, 4, 2, 1):
            scratch[...] = vv
            vv = op(vv, plsc.load_gather(scratch, [iota ^ d]))
        return vv

    def chunk_body(cidx, _):
        seq0 = _P + p0 + cidx * _CHUNK
        pltpu.sync_copy(logits_hbm.at[b, pl.ds(seq0, _CHUNK)], rows_v)

        def row_body(r, _):
            def scan_max(j, carry):
                ms, bjs = carry
                new_ms, new_bjs = [], []
                for q in range(4):
                    xv = rows_v[r, pl.ds((j * 4 + q) * _L, _L)]
                    new_ms.append(jnp.maximum(ms[q], xv))
                    new_bjs.append(jnp.where(xv > ms[q], j, bjs[q]))
                return tuple(new_ms), tuple(new_bjs)

            ms, bjs = jax.lax.fori_loop(
                0, nvec // 4, scan_max,
                (tuple(jnp.full((_L,), -jnp.inf, jnp.float32) for _ in range(4)),
                 tuple(jnp.zeros((_L,), jnp.int32) for _ in range(4))),
                unroll=4)
            m01 = jnp.maximum(ms[0], ms[1])
            m23 = jnp.maximum(ms[2], ms[3])
            gmax = _allred(jnp.maximum(m01, m23), red_f, jnp.maximum)
            cand = jnp.full((_L,), v, jnp.int32)
            for q in range(4):
                idxq = (bjs[q] * 4 + q) * _L + iota
                cand = jnp.minimum(cand, jnp.where(ms[q] == gmax, idxq, v))
            bidx = _allred(cand, red_i, jnp.minimum)

            def scan_sum(j, accs):
                new = []
                for q in range(4):
                    xv = rows_v[r, pl.ds((j * 4 + q) * _L, _L)]
                    new.append(accs[q] + jnp.exp(xv - gmax))
                return tuple(new)

            accs = jax.lax.fori_loop(
                0, nvec // 4, scan_sum,
                tuple(jnp.zeros((_L,), jnp.float32) for _ in range(4)),
                unroll=4)
            ssum = _allred((accs[0] + accs[1]) + (accs[2] + accs[3]),
                           red_f, jnp.add)
            lane0 = iota == 0
            i_spl = jnp.zeros((_L,), jnp.int32) + (cidx * _CHUNK + r)
            plsc.store_scatter(pstage_v, [i_spl], 1.0 / ssum, mask=lane0)
            plsc.store_scatter(tstage_v, [i_spl], bidx, mask=lane0)
            return 0

        jax.lax.fori_loop(0, _CHUNK, row_body, 0)
        return 0

    jax.lax.fori_loop(0, rows_per_worker // _CHUNK, chunk_body, 0)
    bo = b - (8 - _BSC)
    pltpu.sync_copy(pstage_v, pmax_hbm.at[bo, pl.ds(p0, rows_per_worker)])
    pltpu.sync_copy(tstage_v, tok_hbm.at[bo, pl.ds(p0, rows_per_worker)])


def _topk_kernel(ptc_ref, psc_ref, ttc_ref, tsc_ref, shift_ref, x_ref,
                 probs_ref, out_ref):
    p = jnp.concatenate([ptc_ref[...][:, :, 0], psc_ref[...]], axis=0)
    tok = jnp.concatenate([ttc_ref[...][:, :, 0], tsc_ref[...]], axis=0)
    shift = shift_ref[0, 0]
    b, ssuf = p.shape
    iota = jax.lax.broadcasted_iota(jnp.int32, p.shape, 1)
    colk = jax.lax.broadcasted_iota(jnp.int32, (b, _K), 1)
    sel = jnp.zeros((b, _K), jnp.int32)
    vals = jnp.zeros((b, _K), jnp.float32)
    for i in range(_K):
        m = jnp.max(p, axis=1, keepdims=True)
        cand = jnp.where(p == m, iota, ssuf)
        idx = jnp.min(cand, axis=1, keepdims=True)
        sel = jnp.where(colk == i, idx, sel)
        vals = jnp.where(colk == i, m, vals)
        p = jnp.where(iota == idx, -jnp.inf, p)
    probs_ref[...] = vals
    q = sel + shift
    rank = jnp.zeros_like(q)
    for j in range(_K):
        rank = rank + (q[:, j : j + 1] < q).astype(jnp.int32)
    out = jnp.full(p.shape, _MASK_TOKEN_ID, jnp.int32)
    for i in range(_K):
        pos = q[:, i : i + 1]
        t = jnp.sum(jnp.where(iota == pos, tok, 0), axis=1, keepdims=True)
        out = jnp.where(iota == rank[:, i : i + 1], t, out)
    out_ref[:, :_P] = x_ref[:, :_P]
    out_ref[:, _P:] = out


def kernel(logits, x, output_start_idx, k):
    b, s, v = logits.shape
    ssuf = s - _P
    btc = b - _BSC
    cs = 512
    pmax_tc, tok_tc = pl.pallas_call(
        _stats_kernel,
        grid=(btc, ssuf // cs),
        in_specs=[pl.BlockSpec((1, cs, v), lambda i, c: (i, c + _P // cs, 0))],
        out_specs=[
            pl.BlockSpec((1, cs, 1), lambda i, c: (i, c, 0)),
            pl.BlockSpec((1, cs, 1), lambda i, c: (i, c, 0)),
        ],
        out_shape=[
            jax.ShapeDtypeStruct((btc, ssuf, 1), jnp.float32),
            jax.ShapeDtypeStruct((btc, ssuf, 1), jnp.int32),
        ],
    )(logits)
    rows_per_worker = (_BSC * ssuf) // 32
    sc_stats = pl.kernel(
        _sc_stats_kernel,
        out_type=[
            jax.ShapeDtypeStruct((_BSC, ssuf), jnp.float32),
            jax.ShapeDtypeStruct((_BSC, ssuf), jnp.int32),
        ],
        scratch_types=[
            pltpu.VMEM((_CHUNK, v), jnp.float32),
            pltpu.VMEM((rows_per_worker,), jnp.float32),
            pltpu.VMEM((rows_per_worker,), jnp.int32),
            pltpu.VMEM((_L,), jnp.float32),
            pltpu.VMEM((_L,), jnp.int32),
        ],
        mesh=plsc.VectorSubcoreMesh(core_axis_name="c", subcore_axis_name="s"),
        compiler_params=pltpu.CompilerParams(needs_layout_passes=False),
    )
    pmax_sc, tok_sc = sc_stats(logits)
    shift = (jnp.asarray(output_start_idx, jnp.int32) - _P
             + jnp.asarray(k, jnp.int32) - _K).reshape(1, 1)
    probs, out = pl.pallas_call(
        _topk_kernel,
        out_shape=[
            jax.ShapeDtypeStruct((b, _K), jnp.float32),
            jax.ShapeDtypeStruct((b, s), jnp.int32),
        ],
    )(pmax_tc, pmax_sc, tok_tc, tok_sc, shift, x)
    return out, probs


# speedup vs baseline: 2.8682x; 1.0278x over previous
import jax
import jax.numpy as jnp
from jax.experimental import pallas as pl
from jax.experimental.pallas import tpu as pltpu
from jax.experimental.pallas import tpu_sc as plsc

_MASK_TOKEN_ID = 8191
_P = 1024
_K = 16
_L = 16
_BSC = 2
_CHUNK = 4


def _stats_kernel(x_ref, pmax_ref, tok_ref):
    xb = x_ref[0]
    m = jnp.max(xb, axis=1, keepdims=True)
    e = jnp.exp(xb - m)
    s = jnp.sum(e, axis=1, keepdims=True)
    iota = jax.lax.broadcasted_iota(jnp.int32, xb.shape, 1)
    a = jnp.min(jnp.where(xb == m, iota, xb.shape[1]), axis=1, keepdims=True)
    pmax_ref[0] = 1.0 / s
    tok_ref[0] = a


def _sc_stats_kernel(logits_hbm, pmax_hbm, tok_hbm, buf0_v, buf1_v, pstage_v,
                     tstage_v, red_f, red_i, sem0, sem1):
    _, s, v = logits_hbm.shape
    nvec = v // _L
    rows_per_worker = (_BSC * (s - _P)) // 32
    wid = jax.lax.axis_index("s") * 2 + jax.lax.axis_index("c")
    b = (8 - _BSC) + wid // (32 // _BSC)
    p0 = (wid % (32 // _BSC)) * rows_per_worker
    iota = jax.lax.iota(jnp.int32, _L)

    def _allred(vv, scratch, op):
        for d in (8, 4, 2, 1):
            scratch[...] = vv
            vv = op(vv, plsc.load_gather(scratch, [iota ^ d]))
        return vv

    def _src(c):
        return logits_hbm.at[b, pl.ds(_P + p0 + c * _CHUNK, _CHUNK)]

    def _process(rows_v, cidx):
        def row_body(r, _):
            def scan_max(j, carry):
                ms, bjs = carry
                new_ms, new_bjs = [], []
                for q in range(4):
                    xv = rows_v[r, pl.ds((j * 4 + q) * _L, _L)]
                    new_ms.append(jnp.maximum(ms[q], xv))
                    new_bjs.append(jnp.where(xv > ms[q], j, bjs[q]))
                return tuple(new_ms), tuple(new_bjs)

            ms, bjs = jax.lax.fori_loop(
                0, nvec // 4, scan_max,
                (tuple(jnp.full((_L,), -jnp.inf, jnp.float32) for _ in range(4)),
                 tuple(jnp.zeros((_L,), jnp.int32) for _ in range(4))),
                unroll=4)
            m01 = jnp.maximum(ms[0], ms[1])
            m23 = jnp.maximum(ms[2], ms[3])
            gmax = _allred(jnp.maximum(m01, m23), red_f, jnp.maximum)
            cand = jnp.full((_L,), v, jnp.int32)
            for q in range(4):
                idxq = (bjs[q] * 4 + q) * _L + iota
                cand = jnp.minimum(cand, jnp.where(ms[q] == gmax, idxq, v))
            bidx = _allred(cand, red_i, jnp.minimum)

            def scan_sum(j, accs):
                new = []
                for q in range(4):
                    xv = rows_v[r, pl.ds((j * 4 + q) * _L, _L)]
                    new.append(accs[q] + jnp.exp(xv - gmax))
                return tuple(new)

            accs = jax.lax.fori_loop(
                0, nvec // 4, scan_sum,
                tuple(jnp.zeros((_L,), jnp.float32) for _ in range(4)),
                unroll=4)
            ssum = _allred((accs[0] + accs[1]) + (accs[2] + accs[3]),
                           red_f, jnp.add)
            lane0 = iota == 0
            i_spl = jnp.zeros((_L,), jnp.int32) + (cidx * _CHUNK + r)
            plsc.store_scatter(pstage_v, [i_spl], 1.0 / ssum, mask=lane0)
            plsc.store_scatter(tstage_v, [i_spl], bidx, mask=lane0)
            return 0

        jax.lax.fori_loop(0, _CHUNK, row_body, 0)

    nch = rows_per_worker // _CHUNK
    pltpu.make_async_copy(_src(0), buf0_v, sem0).start()

    def pair_body(ci2, _):
        c0 = 2 * ci2
        c1 = c0 + 1
        pltpu.make_async_copy(_src(c1), buf1_v, sem1).start()
        pltpu.make_async_copy(_src(c0), buf0_v, sem0).wait()
        _process(buf0_v, c0)

        @pl.when(c0 + 2 < nch)
        def _():
            pltpu.make_async_copy(_src(c0 + 2), buf0_v, sem0).start()

        pltpu.make_async_copy(_src(c1), buf1_v, sem1).wait()
        _process(buf1_v, c1)
        return 0

    jax.lax.fori_loop(0, nch // 2, pair_body, 0)
    bo = b - (8 - _BSC)
    pltpu.sync_copy(pstage_v, pmax_hbm.at[bo, pl.ds(p0, rows_per_worker)])
    pltpu.sync_copy(tstage_v, tok_hbm.at[bo, pl.ds(p0, rows_per_worker)])


def _topk_kernel(ptc_ref, psc_ref, ttc_ref, tsc_ref, shift_ref, x_ref,
                 probs_ref, out_ref):
    p = jnp.concatenate([ptc_ref[...][:, :, 0], psc_ref[...]], axis=0)
    tok = jnp.concatenate([ttc_ref[...][:, :, 0], tsc_ref[...]], axis=0)
    shift = shift_ref[0, 0]
    b, ssuf = p.shape
    iota = jax.lax.broadcasted_iota(jnp.int32, p.shape, 1)
    colk = jax.lax.broadcasted_iota(jnp.int32, (b, _K), 1)
    sel = jnp.zeros((b, _K), jnp.int32)
    vals = jnp.zeros((b, _K), jnp.float32)
    for i in range(_K):
        m = jnp.max(p, axis=1, keepdims=True)
        cand = jnp.where(p == m, iota, ssuf)
        idx = jnp.min(cand, axis=1, keepdims=True)
        sel = jnp.where(colk == i, idx, sel)
        vals = jnp.where(colk == i, m, vals)
        p = jnp.where(iota == idx, -jnp.inf, p)
    probs_ref[...] = vals
    q = sel + shift
    rank = jnp.zeros_like(q)
    for j in range(_K):
        rank = rank + (q[:, j : j + 1] < q).astype(jnp.int32)
    out = jnp.full(p.shape, _MASK_TOKEN_ID, jnp.int32)
    for i in range(_K):
        pos = q[:, i : i + 1]
        t = jnp.sum(jnp.where(iota == pos, tok, 0), axis=1, keepdims=True)
        out = jnp.where(iota == rank[:, i : i + 1], t, out)
    out_ref[:, :_P] = x_ref[:, :_P]
    out_ref[:, _P:] = out


def kernel(logits, x, output_start_idx, k):
    b, s, v = logits.shape
    ssuf = s - _P
    btc = b - _BSC
    cs = 512
    pmax_tc, tok_tc = pl.pallas_call(
        _stats_kernel,
        grid=(btc, ssuf // cs),
        in_specs=[pl.BlockSpec((1, cs, v), lambda i, c: (i, c + _P // cs, 0))],
        out_specs=[
            pl.BlockSpec((1, cs, 1), lambda i, c: (i, c, 0)),
            pl.BlockSpec((1, cs, 1), lambda i, c: (i, c, 0)),
        ],
        out_shape=[
            jax.ShapeDtypeStruct((btc, ssuf, 1), jnp.float32),
            jax.ShapeDtypeStruct((btc, ssuf, 1), jnp.int32),
        ],
    )(logits)
    rows_per_worker = (_BSC * ssuf) // 32
    sc_stats = pl.kernel(
        _sc_stats_kernel,
        out_type=[
            jax.ShapeDtypeStruct((_BSC, ssuf), jnp.float32),
            jax.ShapeDtypeStruct((_BSC, ssuf), jnp.int32),
        ],
        scratch_types=[
            pltpu.VMEM((_CHUNK, v), jnp.float32),
            pltpu.VMEM((_CHUNK, v), jnp.float32),
            pltpu.VMEM((rows_per_worker,), jnp.float32),
            pltpu.VMEM((rows_per_worker,), jnp.int32),
            pltpu.VMEM((_L,), jnp.float32),
            pltpu.VMEM((_L,), jnp.int32),
            pltpu.SemaphoreType.DMA,
            pltpu.SemaphoreType.DMA,
        ],
        mesh=plsc.VectorSubcoreMesh(core_axis_name="c", subcore_axis_name="s"),
        compiler_params=pltpu.CompilerParams(needs_layout_passes=False),
    )
    pmax_sc, tok_sc = sc_stats(logits)
    shift = (jnp.asarray(output_start_idx, jnp.int32) - _P
             + jnp.asarray(k, jnp.int32) - _K).reshape(1, 1)
    probs, out = pl.pallas_call(
        _topk_kernel,
        out_shape=[
            jax.ShapeDtypeStruct((b, _K), jnp.float32),
            jax.ShapeDtypeStruct((b, s), jnp.int32),
        ],
    )(pmax_tc, pmax_sc, tok_tc, tok_sc, shift, x)
    return out, probs


# speedup vs baseline: 2.8727x; 1.0016x over previous
import jax
import jax.numpy as jnp
from jax.experimental import pallas as pl
from jax.experimental.pallas import tpu as pltpu
from jax.experimental.pallas import tpu_sc as plsc

_MASK_TOKEN_ID = 8191
_P = 1024
_K = 16
_L = 16
_BSC = 2
_CHUNK = 4


def _stats_kernel(x_ref, pmax_ref, tok_ref):
    xb = x_ref[0]
    m = jnp.max(xb, axis=1, keepdims=True)
    e = jnp.exp(xb - m)
    s = jnp.sum(e, axis=1, keepdims=True)
    iota = jax.lax.broadcasted_iota(jnp.int32, xb.shape, 1)
    a = jnp.min(jnp.where(xb == m, iota, xb.shape[1]), axis=1, keepdims=True)
    pmax_ref[0] = 1.0 / s
    tok_ref[0] = a


def _sc_stats_kernel(logits_hbm, pmax_hbm, tok_hbm, buf0_v, buf1_v, pstage_v,
                     tstage_v, red_f, red_i, sem0, sem1):
    _, s, v = logits_hbm.shape
    nvec = v // _L
    rows_per_worker = (_BSC * (s - _P)) // 32
    wid = jax.lax.axis_index("s") * 2 + jax.lax.axis_index("c")
    b = (8 - _BSC) + wid // (32 // _BSC)
    p0 = (wid % (32 // _BSC)) * rows_per_worker
    iota = jax.lax.iota(jnp.int32, _L)

    def _allred(vv, scratch, op):
        for d in (8, 4, 2, 1):
            scratch[...] = vv
            vv = op(vv, plsc.load_gather(scratch, [iota ^ d]))
        return vv

    def _src(c):
        return logits_hbm.at[b, pl.ds(_P + p0 + c * _CHUNK, _CHUNK)]

    def _process(rows_v, cidx):
        def row_body(r, _):
            def scan_max(j, carry):
                ms, bjs = carry
                new_ms, new_bjs = [], []
                for q in range(4):
                    xv = rows_v[r, pl.ds((j * 4 + q) * _L, _L)]
                    new_ms.append(jnp.maximum(ms[q], xv))
                    new_bjs.append(jnp.where(xv > ms[q], j, bjs[q]))
                return tuple(new_ms), tuple(new_bjs)

            ms, bjs = jax.lax.fori_loop(
                0, nvec // 4, scan_max,
                (tuple(jnp.full((_L,), -jnp.inf, jnp.float32) for _ in range(4)),
                 tuple(jnp.zeros((_L,), jnp.int32) for _ in range(4))),
                unroll=4)
            m01 = jnp.maximum(ms[0], ms[1])
            m23 = jnp.maximum(ms[2], ms[3])
            gmax = _allred(jnp.maximum(m01, m23), red_f, jnp.maximum)
            cand = jnp.full((_L,), v, jnp.int32)
            for q in range(4):
                idxq = (bjs[q] * 4 + q) * _L + iota
                cand = jnp.minimum(cand, jnp.where(ms[q] == gmax, idxq, v))
            bidx = _allred(cand, red_i, jnp.minimum)

            def scan_sum(j, accs):
                new = []
                for q in range(4):
                    xv = rows_v[r, pl.ds((j * 4 + q) * _L, _L)]
                    new.append(accs[q] + jnp.exp(xv - gmax))
                return tuple(new)

            accs = jax.lax.fori_loop(
                0, nvec // 4, scan_sum,
                tuple(jnp.zeros((_L,), jnp.float32) for _ in range(4)),
                unroll=4)
            ssum = _allred((accs[0] + accs[1]) + (accs[2] + accs[3]),
                           red_f, jnp.add)
            lane0 = iota == 0
            i_spl = jnp.zeros((_L,), jnp.int32) + (cidx * _CHUNK + r)
            plsc.store_scatter(pstage_v, [i_spl], 1.0 / ssum, mask=lane0)
            plsc.store_scatter(tstage_v, [i_spl], bidx, mask=lane0)
            return 0

        jax.lax.fori_loop(0, _CHUNK, row_body, 0)

    nch = rows_per_worker // _CHUNK
    pltpu.make_async_copy(_src(0), buf0_v, sem0).start()

    def pair_body(ci2, _):
        c0 = 2 * ci2
        c1 = c0 + 1
        pltpu.make_async_copy(_src(c1), buf1_v, sem1).start()
        pltpu.make_async_copy(_src(c0), buf0_v, sem0).wait()
        _process(buf0_v, c0)

        @pl.when(c0 + 2 < nch)
        def _():
            pltpu.make_async_copy(_src(c0 + 2), buf0_v, sem0).start()

        pltpu.make_async_copy(_src(c1), buf1_v, sem1).wait()
        _process(buf1_v, c1)
        return 0

    jax.lax.fori_loop(0, nch // 2, pair_body, 0)
    bo = b - (8 - _BSC)
    pltpu.sync_copy(pstage_v, pmax_hbm.at[bo, pl.ds(p0, rows_per_worker)])
    pltpu.sync_copy(tstage_v, tok_hbm.at[bo, pl.ds(p0, rows_per_worker)])


def _topk_kernel(ptc_ref, psc_ref, ttc_ref, tsc_ref, shift_ref, x_ref,
                 probs_ref, out_ref):
    p = jnp.concatenate([ptc_ref[...][:, :, 0], psc_ref[...]], axis=0)
    tok = jnp.concatenate([ttc_ref[...][:, :, 0], tsc_ref[...]], axis=0)
    shift = shift_ref[0, 0]
    b, ssuf = p.shape
    iota = jax.lax.broadcasted_iota(jnp.int32, p.shape, 1)
    colk = jax.lax.broadcasted_iota(jnp.int32, (b, _K), 1)
    sel = jnp.zeros((b, _K), jnp.int32)
    vals = jnp.zeros((b, _K), jnp.float32)
    for i in range(_K):
        m = jnp.max(p, axis=1, keepdims=True)
        cand = jnp.where(p == m, iota, ssuf)
        idx = jnp.min(cand, axis=1, keepdims=True)
        sel = jnp.where(colk == i, idx, sel)
        vals = jnp.where(colk == i, m, vals)
        p = jnp.where(iota == idx, -jnp.inf, p)
    probs_ref[...] = vals
    q = sel + shift
    rank = jnp.zeros_like(q)
    for j in range(_K):
        rank = rank + (q[:, j : j + 1] < q).astype(jnp.int32)
    out = jnp.full(p.shape, _MASK_TOKEN_ID, jnp.int32)
    for i in range(_K):
        pos = q[:, i : i + 1]
        t = jnp.sum(jnp.where(iota == pos, tok, 0), axis=1, keepdims=True)
        out = jnp.where(iota == rank[:, i : i + 1], t, out)
    out_ref[:, :_P] = x_ref[:, :_P]
    out_ref[:, _P:] = out


def kernel(logits, x, output_start_idx, k):
    b, s, v = logits.shape
    ssuf = s - _P
    btc = b - _BSC
    cs = 512
    rows_per_worker = (_BSC * ssuf) // 32
    sc_stats = pl.kernel(
        _sc_stats_kernel,
        out_type=[
            jax.ShapeDtypeStruct((_BSC, ssuf), jnp.float32),
            jax.ShapeDtypeStruct((_BSC, ssuf), jnp.int32),
        ],
        scratch_types=[
            pltpu.VMEM((_CHUNK, v), jnp.float32),
            pltpu.VMEM((_CHUNK, v), jnp.float32),
            pltpu.VMEM((rows_per_worker,), jnp.float32),
            pltpu.VMEM((rows_per_worker,), jnp.int32),
            pltpu.VMEM((_L,), jnp.float32),
            pltpu.VMEM((_L,), jnp.int32),
            pltpu.SemaphoreType.DMA,
            pltpu.SemaphoreType.DMA,
        ],
        mesh=plsc.VectorSubcoreMesh(core_axis_name="c", subcore_axis_name="s"),
        compiler_params=pltpu.CompilerParams(needs_layout_passes=False),
    )
    pmax_sc, tok_sc = sc_stats(logits)
    pmax_tc, tok_tc = pl.pallas_call(
        _stats_kernel,
        grid=(btc, ssuf // cs),
        in_specs=[pl.BlockSpec((1, cs, v), lambda i, c: (i, c + _P // cs, 0))],
        out_specs=[
            pl.BlockSpec((1, cs, 1), lambda i, c: (i, c, 0)),
            pl.BlockSpec((1, cs, 1), lambda i, c: (i, c, 0)),
        ],
        out_shape=[
            jax.ShapeDtypeStruct((btc, ssuf, 1), jnp.float32),
            jax.ShapeDtypeStruct((btc, ssuf, 1), jnp.int32),
        ],
    )(logits)
    shift = (jnp.asarray(output_start_idx, jnp.int32) - _P
             + jnp.asarray(k, jnp.int32) - _K).reshape(1, 1)
    probs, out = pl.pallas_call(
        _topk_kernel,
        out_shape=[
            jax.ShapeDtypeStruct((b, _K), jnp.float32),
            jax.ShapeDtypeStruct((b, s), jnp.int32),
        ],
    )(pmax_tc, pmax_sc, tok_tc, tok_sc, shift, x)
    return out, probs
